# Initial kernel scaffold; baseline (speedup 1.0000x reference)
import jax
import jax.numpy as jnp
from jax.experimental import pallas as pl


def kernel(sequences, lengths, mb, probs_x, probs_y, scale):
    raise NotImplementedError("write your pallas kernel here")


# speedup vs baseline: 8.5282x; 8.5282x over previous
import functools

import jax
import jax.numpy as jnp
from jax.experimental import pallas as pl
from jax.experimental.pallas import tpu as pltpu

_HIGH = jax.lax.Precision.HIGHEST


def _fwd_kernel(mb_ref, seq_ref, px_ref, py_ref, lens_ref, out_ref, emit_ref,
                *, num_b, seq_len, chunk):
    i = pl.program_id(0)

    py = py_ref[...]
    l1mpy = jnp.log1p(-py)
    w = jnp.log(py) - l1mpy
    bias = jnp.sum(l1mpy, axis=1)

    s = seq_ref[0]
    e = jax.lax.dot_general(s, w, (((1,), (1,)), ((), ())),
                            preferred_element_type=jnp.float32,
                            precision=_HIGH) + bias[None, :]
    emit_ref[i] = e

    @pl.when(i == num_b - 1)
    def _scan():
        px = px_ref[...]
        lens = lens_ref[...]
        lpx0 = jnp.log(px[0:1, :])
        alpha0 = lpx0 + emit_ref[:, 0, :]

        def chunk_body(k, alpha):
            blk = emit_ref[:, pl.ds(k * chunk, chunk), :]
            for j in range(chunk):
                t = k * chunk + j
                m = jnp.max(alpha, axis=1, keepdims=True)
                p = jnp.exp(alpha - m)
                sdot = jax.lax.dot_general(p, px, (((1,), (0,)), ((), ())),
                                           preferred_element_type=jnp.float32,
                                           precision=_HIGH)
                new = m + jnp.log(sdot) + blk[:, j, :]
                mask = (t >= 1) & (t < lens)
                alpha = jnp.where(mask, new, alpha)
            return alpha

        alpha = jax.lax.fori_loop(0, seq_len // chunk, chunk_body, alpha0)

        m2 = jnp.max(alpha, axis=1, keepdims=True)
        ll = m2 + jnp.log(jnp.sum(jnp.exp(alpha - m2), axis=1, keepdims=True))
        out_ref[...] = jnp.sum(ll, axis=0, keepdims=True)


def kernel(sequences, lengths, mb, probs_x, probs_y, scale=1.0):
    num_seq, seq_len, data_dim = sequences.shape
    hidden = probs_x.shape[0]
    num_b = mb.shape[0]
    chunk = 8

    lens = lengths[mb].reshape(num_b, 1)

    grid_spec = pltpu.PrefetchScalarGridSpec(
        num_scalar_prefetch=1,
        grid=(num_b,),
        in_specs=[
            pl.BlockSpec((1, seq_len, data_dim), lambda i, mb_ref: (mb_ref[i], 0, 0)),
            pl.BlockSpec((hidden, hidden), lambda i, mb_ref: (0, 0)),
            pl.BlockSpec((hidden, data_dim), lambda i, mb_ref: (0, 0)),
            pl.BlockSpec((num_b, 1), lambda i, mb_ref: (0, 0)),
        ],
        out_specs=pl.BlockSpec((1, 1), lambda i, mb_ref: (0, 0)),
        scratch_shapes=[pltpu.VMEM((num_b, seq_len, hidden), jnp.float32)],
    )

    out = pl.pallas_call(
        functools.partial(_fwd_kernel, num_b=num_b, seq_len=seq_len, chunk=chunk),
        grid_spec=grid_spec,
        out_shape=jax.ShapeDtypeStruct((1, 1), jnp.float32),
    )(mb, sequences, probs_x, probs_y, lens)

    return (scale * out[0, 0]).astype(jnp.float32)
